# baseline (speedup 1.0000x reference)
import functools

import jax
import jax.numpy as jnp
from jax import lax
from jax.experimental import pallas as pl
from jax.experimental.pallas import tpu as pltpu
from jax.experimental.pallas import tpu_sc as plsc

N, D, B = 1000000, 64, 16384
NP = N // 2
NC, NS = 2, 16
NW = NC * NS
RPW = B // NW
K = 64
NCHUNK = RPW // K
G = K // 16


def _body(inp2, mem2, sp, dp, dn, ds, dn0, o1, o2, o3, o4, o5,
          sp_v, dp_v, dn_v, ds_v, dn0_v,
          pr_a, pr_b, stg_a, stg_b, gs_a, gs_b, ss_a, ss_b):
    wid = lax.axis_index("s") * NC + lax.axis_index("c")
    base = wid * RPW
    cbase = wid * NCHUNK
    cp = pltpu.sync_copy
    cp(sp.at[pl.ds(cbase, NCHUNK)], sp_v)
    cp(dp.at[pl.ds(cbase, NCHUNK)], dp_v)
    cp(dn.at[pl.ds(cbase, NCHUNK)], dn_v)
    cp(ds.at[pl.ds(cbase, NCHUNK)], ds_v)
    cp(dn0.at[pl.ds(cbase, NCHUNK)], dn0_v)

    def fire_gathers(tbl, gidx_v, c, prs, gsem):
        def grp(g, carry):
            r16 = gidx_v[c, pl.ds(g * 16, 16)]
            t16 = lax.shift_right_logical(r16, 1)
            for u in range(16):
                pltpu.async_copy(tbl.at[t16[u]], prs.at[g * 16 + u], gsem)
            return carry
        lax.fori_loop(0, G, grp, 0)

    def drain_gathers(prs, gsem):
        pltpu.make_async_copy(inp2.at[pl.ds(0, K)], prs, gsem).wait()

    def extract(gidx_v, c, prs, stg):
        def grp(g, carry):
            r16 = gidx_v[c, pl.ds(g * 16, 16)]
            h16 = lax.shift_left(lax.bitwise_and(r16, 1), 6)
            for u in range(16):
                j = g * 16 + u
                h = h16[u]
                for q in range(D // 16):
                    stg[j, pl.ds(q * 16, 16)] = prs[j, pl.ds(h + q * 16, 16)]
            return carry
        lax.fori_loop(0, G, grp, 0)

    def fire_scatter(out, sidx_v, c, stg, ssem):
        if sidx_v is None:
            return pltpu.async_copy(
                stg, out.at[pl.ds(base + c * K, K)], ssem)
        return pltpu.async_copy(stg, out.at[sidx_v.at[c]], ssem)

    def task(tbl, gidx_v, out, sidx_v):
        def pair(k, carry):
            c0 = 2 * k
            c1 = 2 * k + 1
            fire_gathers(tbl, gidx_v, c0, pr_a, gs_a)
            fire_gathers(tbl, gidx_v, c1, pr_b, gs_b)
            drain_gathers(pr_a, gs_a)
            extract(gidx_v, c0, pr_a, stg_a)
            sa = fire_scatter(out, sidx_v, c0, stg_a, ss_a)
            drain_gathers(pr_b, gs_b)
            extract(gidx_v, c1, pr_b, stg_b)
            sb = fire_scatter(out, sidx_v, c1, stg_b, ss_b)
            sa.wait()
            sb.wait()
            return carry
        lax.fori_loop(0, NCHUNK // 2, pair, 0)

    task(inp2, sp_v, o1, None)
    task(inp2, dp_v, o2, ds_v)
    task(inp2, dn_v, o3, dn0_v)
    task(mem2, dp_v, o4, ds_v)
    task(mem2, sp_v, o5, ds_v)


_out = jax.ShapeDtypeStruct((B, 128), jnp.float32)
_sc_call = functools.partial(
    pl.kernel,
    out_type=(_out,) * 5,
    mesh=plsc.VectorSubcoreMesh(core_axis_name="c", subcore_axis_name="s"),
    scratch_types=[
        pltpu.VMEM((NCHUNK, K), jnp.int32),
        pltpu.VMEM((NCHUNK, K), jnp.int32),
        pltpu.VMEM((NCHUNK, K), jnp.int32),
        pltpu.VMEM((NCHUNK, K), jnp.int32),
        pltpu.VMEM((NCHUNK, K), jnp.int32),
        pltpu.VMEM((K, 128), jnp.float32),
        pltpu.VMEM((K, 128), jnp.float32),
        pltpu.VMEM((K, 128), jnp.float32),
        pltpu.VMEM((K, 128), jnp.float32),
        pltpu.SemaphoreType.DMA,
        pltpu.SemaphoreType.DMA,
        pltpu.SemaphoreType.DMA,
        pltpu.SemaphoreType.DMA,
    ],
    compiler_params=pltpu.CompilerParams(use_tc_tiling_on_sc=True,
                                         needs_layout_passes=False),
)(_body)


def kernel(input, memory, src_pos_index, dst_pos_index, dst_neg_index,
           dist_src_index, dist_neg_src_index, neg_samples):
    del neg_samples
    inp2 = input.reshape(NP, 2 * D)
    mem2 = memory.reshape(NP, 2 * D)
    sp = src_pos_index.reshape(B // K, K)
    dp = dst_pos_index.reshape(B // K, K)
    dn = dst_neg_index.reshape(B // K, K)
    ds = dist_src_index.reshape(B // K, K)
    dn0 = (dist_neg_src_index - B).reshape(B // K, K)
    o1, o2, o3, o4, o5 = _sc_call(inp2, mem2, sp, dp, dn, ds, dn0)
    return (o1[:, :D], o2[:, :D], o3[:, :D], o4[:, :D], o5[:, :D])


# speedup vs baseline: 2.3401x; 2.3401x over previous
import functools

import jax
import jax.numpy as jnp
from jax import lax
from jax.experimental import pallas as pl
from jax.experimental.pallas import tpu as pltpu
from jax.experimental.pallas import tpu_sc as plsc

N, D, B = 1000000, 64, 16384
NT = N // 8
NC, NS = 2, 16
NW = NC * NS
RPW = B // NW
K = 128
NCHUNK = RPW // K
G = K // 16


def _body(tbl_i, tbl_m, sp, dp, dn, ds, dn0, o1, o2, o3, o4, o5,
          sp_v, dp_v, dn_v, ds_v, dn0_v, stg_a, stg_b,
          gs_a, gs_b, ss_a, ss_b):
    wid = lax.axis_index("s") * NC + lax.axis_index("c")
    base = wid * RPW
    cbase = wid * NCHUNK
    cp = pltpu.sync_copy
    cp(sp.at[pl.ds(cbase, NCHUNK)], sp_v)
    cp(dp.at[pl.ds(cbase, NCHUNK)], dp_v)
    cp(dn.at[pl.ds(cbase, NCHUNK)], dn_v)
    cp(ds.at[pl.ds(cbase, NCHUNK)], ds_v)
    cp(dn0.at[pl.ds(cbase, NCHUNK)], dn0_v)

    def fire_gathers(tbl, gidx_v, c, stg, gsem):
        def grp(g, carry):
            r16 = gidx_v[c, pl.ds(g * 16, 16)]
            t16 = lax.shift_right_logical(r16, 3)
            s16 = lax.bitwise_and(r16, 7)
            for u in range(16):
                pltpu.async_copy(tbl.at[t16[u], s16[u]],
                                 stg.at[g * 16 + u, pl.ds(0, D)], gsem)
            return carry
        lax.fori_loop(0, G, grp, 0)

    def drain_gathers(stg, gsem):
        pltpu.make_async_copy(o1.at[pl.ds(0, K // 2)],
                              stg.at[pl.ds(0, K // 2)], gsem).wait()

    def fire_scatter(out, sidx_v, c, stg, ssem):
        if sidx_v is None:
            return pltpu.async_copy(
                stg, out.at[pl.ds(base + c * K, K)], ssem)
        return pltpu.async_copy(stg, out.at[sidx_v.at[c]], ssem)

    def task(tbl, gidx_v, out, sidx_v):
        def pair(k, carry):
            c0 = 2 * k
            c1 = 2 * k + 1
            fire_gathers(tbl, gidx_v, c0, stg_a, gs_a)
            fire_gathers(tbl, gidx_v, c1, stg_b, gs_b)
            drain_gathers(stg_a, gs_a)
            sa = fire_scatter(out, sidx_v, c0, stg_a, ss_a)
            drain_gathers(stg_b, gs_b)
            sb = fire_scatter(out, sidx_v, c1, stg_b, ss_b)
            sa.wait()
            sb.wait()
            return carry
        lax.fori_loop(0, NCHUNK // 2, pair, 0)

    task(tbl_i, sp_v, o1, None)
    task(tbl_i, dp_v, o2, ds_v)
    task(tbl_i, dn_v, o3, dn0_v)
    task(tbl_m, dp_v, o4, ds_v)
    task(tbl_m, sp_v, o5, ds_v)


_out = jax.ShapeDtypeStruct((B, 128), jnp.float32)
_sc_call = functools.partial(
    pl.kernel,
    out_type=(_out,) * 5,
    mesh=plsc.VectorSubcoreMesh(core_axis_name="c", subcore_axis_name="s"),
    scratch_types=[
        pltpu.VMEM((NCHUNK, K), jnp.int32),
        pltpu.VMEM((NCHUNK, K), jnp.int32),
        pltpu.VMEM((NCHUNK, K), jnp.int32),
        pltpu.VMEM((NCHUNK, K), jnp.int32),
        pltpu.VMEM((NCHUNK, K), jnp.int32),
        pltpu.VMEM((K, 128), jnp.float32),
        pltpu.VMEM((K, 128), jnp.float32),
        pltpu.SemaphoreType.DMA,
        pltpu.SemaphoreType.DMA,
        pltpu.SemaphoreType.DMA,
        pltpu.SemaphoreType.DMA,
    ],
    compiler_params=pltpu.CompilerParams(use_tc_tiling_on_sc=True,
                                         needs_layout_passes=False),
)(_body)


def kernel(input, memory, src_pos_index, dst_pos_index, dst_neg_index,
           dist_src_index, dist_neg_src_index, neg_samples):
    del neg_samples
    sp = src_pos_index.reshape(B // K, K)
    dp = dst_pos_index.reshape(B // K, K)
    dn = dst_neg_index.reshape(B // K, K)
    ds = dist_src_index.reshape(B // K, K)
    dn0 = (dist_neg_src_index - B).reshape(B // K, K)
    o1, o2, o3, o4, o5 = _sc_call(input.reshape(NT, 8, D),
                                  memory.reshape(NT, 8, D),
                                  sp, dp, dn, ds, dn0)
    return (o1[:, :D], o2[:, :D], o3[:, :D], o4[:, :D], o5[:, :D])


# speedup vs baseline: 2.3745x; 1.0147x over previous
import functools

import jax
import jax.numpy as jnp
from jax import lax
from jax.experimental import pallas as pl
from jax.experimental.pallas import tpu as pltpu
from jax.experimental.pallas import tpu_sc as plsc

N, D, B = 1000000, 64, 16384
NT = N // 8
NC, NS = 2, 16
NW = NC * NS
RPW = B // NW
K = 128
NCHUNK = RPW // K
G = K // 16


def _make_body(ntask, linear_first):
    def _body(tbl, *refs):
        idx_hbm = refs[:ntask]
        sidx_hbm = refs[ntask:2 * ntask]
        outs = refs[2 * ntask:3 * ntask]
        gidx_vs = refs[3 * ntask:4 * ntask]
        sidx_vs = refs[4 * ntask:5 * ntask]
        stg_a, stg_b, gs_a, gs_b, ss_a, ss_b = refs[5 * ntask:]
        wid = lax.axis_index("s") * NC + lax.axis_index("c")
        base = wid * RPW
        cbase = wid * NCHUNK
        cp = pltpu.sync_copy
        for src, dst in zip(idx_hbm, gidx_vs):
            cp(src.at[pl.ds(cbase, NCHUNK)], dst)
        for src, dst in zip(sidx_hbm, sidx_vs):
            cp(src.at[pl.ds(cbase, NCHUNK)], dst)

        def fire_gathers(gidx_v, c, stg, gsem):
            def grp(g, carry):
                r16 = gidx_v[c, pl.ds(g * 16, 16)]
                t16 = lax.shift_right_logical(r16, 3)
                s16 = lax.bitwise_and(r16, 7)
                for u in range(16):
                    pltpu.async_copy(tbl.at[t16[u], s16[u]],
                                     stg.at[g * 16 + u, pl.ds(0, D)], gsem)
                return carry
            lax.fori_loop(0, G, grp, 0)

        def drain_gathers(stg, gsem):
            pltpu.make_async_copy(outs[0].at[pl.ds(0, K // 2)],
                                  stg.at[pl.ds(0, K // 2)], gsem).wait()

        def fire_scatter(out, sidx_v, c, stg, ssem):
            if sidx_v is None:
                return pltpu.async_copy(
                    stg, out.at[pl.ds(base + c * K, K)], ssem)
            return pltpu.async_copy(stg, out.at[sidx_v.at[c]], ssem)

        def task(gidx_v, out, sidx_v):
            def pair(k, carry):
                c0 = 2 * k
                c1 = 2 * k + 1
                fire_gathers(gidx_v, c0, stg_a, gs_a)
                fire_gathers(gidx_v, c1, stg_b, gs_b)
                drain_gathers(stg_a, gs_a)
                sa = fire_scatter(out, sidx_v, c0, stg_a, ss_a)
                drain_gathers(stg_b, gs_b)
                sb = fire_scatter(out, sidx_v, c1, stg_b, ss_b)
                sa.wait()
                sb.wait()
                return carry
            lax.fori_loop(0, NCHUNK // 2, pair, 0)

        for t in range(ntask):
            linear = linear_first and t == 0
            task(gidx_vs[t], outs[t], None if linear else sidx_vs[t])

    return _body


_out = jax.ShapeDtypeStruct((B, 128), jnp.float32)


def _make_call(ntask, linear_first):
    return functools.partial(
        pl.kernel,
        out_type=(_out,) * ntask,
        mesh=plsc.VectorSubcoreMesh(core_axis_name="c", subcore_axis_name="s"),
        scratch_types=(
            [pltpu.VMEM((NCHUNK, K), jnp.int32)] * (2 * ntask)
            + [pltpu.VMEM((K, 128), jnp.float32)] * 2
            + [pltpu.SemaphoreType.DMA] * 4
        ),
        compiler_params=pltpu.CompilerParams(use_tc_tiling_on_sc=True,
                                             needs_layout_passes=False),
    )(_make_body(ntask, linear_first))


_call_inp = _make_call(3, True)
_call_mem = _make_call(2, False)


def kernel(input, memory, src_pos_index, dst_pos_index, dst_neg_index,
           dist_src_index, dist_neg_src_index, neg_samples):
    del neg_samples
    sp = src_pos_index.reshape(B // K, K)
    dp = dst_pos_index.reshape(B // K, K)
    dn = dst_neg_index.reshape(B // K, K)
    ds = dist_src_index.reshape(B // K, K)
    dn0 = (dist_neg_src_index - B).reshape(B // K, K)
    o1, o2, o3 = _call_inp(input.reshape(NT, 8, D),
                           sp, dp, dn, sp, ds, dn0)
    o4, o5 = _call_mem(memory.reshape(NT, 8, D),
                       dp, sp, ds, ds)
    return (o1[:, :D], o2[:, :D], o3[:, :D], o4[:, :D], o5[:, :D])
